# baseline (speedup 1.0000x reference)
import functools

import jax
import jax.numpy as jnp
from jax import lax
from jax.experimental import pallas as pl
from jax.experimental.pallas import tpu as pltpu
from jax.experimental.pallas import tpu_sc as plsc

B, N, C, K = 8, 2048, 128, 16
TM = 256
TN = 512
NUM_IDX = B * N * K


TQ = 128
_NG = N // 16
_R = 8


def _knn_body(xq_ref, xc_ref, pt_ref, idx_ref, lp_ref, dscr):
    b = pl.program_id(0)
    acc = jnp.dot(xq_ref[0], xc_ref[0],
                  preferred_element_type=jnp.float32,
                  precision=jax.lax.Precision.HIGHEST)
    dscr[...] = acc
    m = jnp.max(dscr[...].reshape(2, _NG, 8, TQ), axis=1)
    pt = pt_ref[...]
    lp_ref[...] = jnp.where(pt >= 0, pt, 0.01 * pt)
    g4 = lax.broadcasted_iota(jnp.int32, (2, _NG, 8, 1), 1)

    vals, ixs = [], []
    for i in range(_R):
        w = dscr[...].reshape(2, _NG, 8, TQ)
        mb = m[:, None]
        eq = w == mb
        ixg = jnp.min(jnp.where(eq, g4, _NG - 1), axis=1)
        vals.append(m)
        ixs.append(ixg)
        if i < _R - 1:
            m = jnp.max(jnp.where(w < mb, w, -jnp.inf), axis=1)

    aio = lax.broadcasted_iota(jnp.int32, (2, 8, TQ), 0)
    sio = lax.broadcasted_iota(jnp.int32, (2, 8, TQ), 1)
    wv = jnp.concatenate([v.reshape(16, TQ) for v in vals], axis=0)
    nn = jnp.concatenate(
        [((aio * _NG + ix) * 8 + sio).reshape(16, TQ) for ix in ixs], axis=0)

    out = jnp.zeros((32, TQ), jnp.int32)
    orow = lax.broadcasted_iota(jnp.int32, (32, TQ), 0)
    base = b * N
    for j in range(K + 1):
        mm = jnp.max(wv, axis=0, keepdims=True)
        eq2 = wv == mm
        sel = jnp.min(jnp.where(eq2, nn, jnp.int32(1 << 30)),
                      axis=0, keepdims=True)
        if j > 0:
            out = jnp.where(orow == j - 1,
                            jnp.broadcast_to(sel + base, (32, TQ)), out)
        if j < K:
            wv = jnp.where(eq2 & (nn == sel), -jnp.inf, wv)
    idx_ref[...] = jnp.transpose(out, (1, 0))[:, :K]


B2 = B // 2


def _knn_call(xc, xq, pT, h):
    b0 = h * B2
    qn = N // TQ
    return pl.pallas_call(
        _knn_body,
        grid=(B2, qn),
        in_specs=[
            pl.BlockSpec((1, N, 8), lambda b, q: (b + b0, 0, 0)),
            pl.BlockSpec((1, 8, TQ), lambda b, q: (b + b0, 0, q)),
            pl.BlockSpec((TQ, C), lambda b, q: ((b + b0) * qn + q, 0)),
        ],
        out_specs=[
            pl.BlockSpec((TQ, K), lambda b, q: (b * qn + q, 0)),
            pl.BlockSpec((TQ, C), lambda b, q: (b * qn + q, 0)),
        ],
        out_shape=[
            jax.ShapeDtypeStruct((B2 * N, K), jnp.int32),
            jax.ShapeDtypeStruct((B2 * N, C), jnp.float32),
        ],
        scratch_shapes=[pltpu.VMEM((N, TQ), jnp.float32)],
    )(xq, xc, pT)


_NW = 32
_NIH = NUM_IDX // 2
_BPW = _NIH // _NW
_CH = 128
_NCHUNK = _BPW // _CH
_PPC = _CH // K
_NB = 4


def _sc_gather_sum(lpT, gidx):
    mesh = plsc.VectorSubcoreMesh(core_axis_name="c", subcore_axis_name="s")

    @functools.partial(
        pl.kernel,
        mesh=mesh,
        out_type=jax.ShapeDtypeStruct((B2 * N, C), jnp.float32),
        scratch_types=[
            pltpu.VMEM((_NB, _CH), jnp.int32),
            pltpu.VMEM((_NB, _CH, C), jnp.float32),
            pltpu.VMEM_SHARED((16, _NB, _PPC, C), jnp.float32),
            pltpu.VMEM((_PPC, C), jnp.float32),
            pltpu.VMEM((_CH,), jnp.int32),
        ] + [pltpu.SemaphoreType.DMA] * (2 * _NB),
    )
    def k(lp_hbm, idx_hbm, out_hbm, idx_v, rows_v, acc_sh, zeros_v, seg_v,
          *sems):
        gsem = sems[:_NB]
        osem = sems[_NB:]
        sid = lax.axis_index("s")
        wid = sid * 2 + lax.axis_index("c")
        ibase = wid * _BPW
        pbase = wid * (_BPW // K)
        for r in range(_PPC):
            seg_v[pl.ds(r * K, K)] = jnp.full((K,), r, jnp.int32)
            for g in range(C // 16):
                zeros_v[r, pl.ds(g * 16, 16)] = jnp.zeros((16,), jnp.float32)
        for s in range(_NB):
            pltpu.sync_copy(idx_hbm.at[pl.ds(ibase + s * _CH, _CH)],
                            idx_v.at[s])
            pltpu.async_copy(lp_hbm.at[idx_v.at[s]], rows_v.at[s], gsem[s])

        @pl.loop(0, _NCHUNK, step=_NB)
        def _(t0):
            for s in range(_NB):
                t = t0 + s
                acc = acc_sh.at[sid, s]
                @pl.when(t >= _NB)
                def _():
                    pltpu.make_async_copy(
                        acc, out_hbm.at[pl.ds(pbase, _PPC)],
                        osem[s]).wait()
                pltpu.sync_copy(zeros_v, acc)
                pltpu.make_async_copy(lp_hbm.at[pl.ds(0, _CH)], rows_v.at[s],
                                      gsem[s]).wait()
                pltpu.sync_copy(rows_v.at[s], acc.at[seg_v], add=True)
                pltpu.async_copy(acc,
                                 out_hbm.at[pl.ds(pbase + t * _PPC, _PPC)],
                                 osem[s])
                @pl.when(t + _NB < _NCHUNK)
                def _():
                    pltpu.sync_copy(
                        idx_hbm.at[pl.ds(ibase + (t + _NB) * _CH, _CH)],
                        idx_v.at[s])
                    pltpu.async_copy(lp_hbm.at[idx_v.at[s]], rows_v.at[s],
                                     gsem[s])

        for s in range(_NB):
            pltpu.make_async_copy(acc_sh.at[sid, s],
                                  out_hbm.at[pl.ds(pbase, _PPC)],
                                  osem[s]).wait()

    return k(lpT, gidx)


def _mm_body(p_ref, ns_ref, w0_ref, w1_ref, w2_ref, w3_ref,
             b0_ref, b1_ref, b2_ref, b3_ref, out_ref):
    p = p_ref[...]
    lp = jnp.where(p >= 0, p, 0.01 * p)
    ns = ns_ref[...]
    t1 = (jnp.dot(lp, w0_ref[...], preferred_element_type=jnp.float32)
          + b0_ref[...]
          + jnp.dot(ns, w1_ref[...], preferred_element_type=jnp.float32)
          + K * b1_ref[...]) * (1.0 / (K + 1)) + p
    lt1 = jnp.where(t1 >= 0, t1, 0.01 * t1)
    w23 = w2_ref[...] + w3_ref[...]
    out_ref[...] = (jnp.dot(lt1, w23, preferred_element_type=jnp.float32)
                    + (b2_ref[...] + b3_ref[...])) * 0.5 + t1


def _mm_call(pT, ns, w0t, w1t, w2t, w3t, b0, b1, b2, b3, h):
    t0 = h * (B2 * N // TN)
    wspec = pl.BlockSpec((C, C), lambda i: (0, 0))
    bspec = pl.BlockSpec((1, C), lambda i: (0, 0))
    return pl.pallas_call(
        _mm_body,
        grid=(B2 * N // TN,),
        in_specs=[
            pl.BlockSpec((TN, C), lambda i: (i + t0, 0)),
            pl.BlockSpec((TN, C), lambda i: (i, 0)),
            wspec, wspec, wspec, wspec,
            bspec, bspec, bspec, bspec,
        ],
        out_specs=pl.BlockSpec((TN, C), lambda i: (i, 0)),
        out_shape=jax.ShapeDtypeStruct((B2 * N, C), jnp.float32),
    )(pT, ns, w0t, w1t, w2t, w3t, b0, b1, b2, b3)


def kernel(xyz, points, W0, b0, W1, b1, W2, b2, W3, b3):
    xt = jnp.transpose(xyz, (0, 2, 1))
    sq = jnp.sum(xt * xt, axis=2, keepdims=True)
    one = jnp.ones((B, N, 1), jnp.float32)
    zero = jnp.zeros((B, N, 3), jnp.float32)
    xq = jnp.concatenate([xt, sq, one, zero], axis=2)
    xc = jnp.transpose(
        jnp.concatenate([-2.0 * xt, one, sq, zero], axis=2), (0, 2, 1))
    pT = jnp.transpose(points, (0, 2, 1)).reshape(B * N, C)
    args = (W0.T, W1.T, W2.T, W3.T, b0.reshape(1, C), b1.reshape(1, C),
            b2.reshape(1, C), b3.reshape(1, C))
    outs = []
    for h in range(2):
        idx_h, lp_h = _knn_call(xc, xq, pT, h)
        ns_h = _sc_gather_sum(lp_h, idx_h.reshape(_NIH))
        outs.append(_mm_call(pT, ns_h, *args, h))
    outT = jnp.concatenate(outs, axis=0)
    return jnp.transpose(outT.reshape(B, N, C), (0, 2, 1))


# speedup vs baseline: 1.0196x; 1.0196x over previous
import functools

import jax
import jax.numpy as jnp
from jax import lax
from jax.experimental import pallas as pl
from jax.experimental.pallas import tpu as pltpu
from jax.experimental.pallas import tpu_sc as plsc

B, N, C, K = 8, 2048, 128, 16
TM = 256
TN = 512
NUM_IDX = B * N * K


TQ = 128
_NG = N // 16
_R = 8


def _knn_body(xq_ref, xc_ref, pt_ref, idx_ref, lp_ref, dscr):
    b = pl.program_id(0)
    acc = jnp.dot(xq_ref[0], xc_ref[0],
                  preferred_element_type=jnp.float32,
                  precision=jax.lax.Precision.HIGHEST)
    dscr[...] = acc
    m = jnp.max(dscr[...].reshape(2, _NG, 8, TQ), axis=1)
    pt = pt_ref[...]
    lp_ref[...] = jnp.where(pt >= 0, pt, 0.01 * pt)
    g4 = lax.broadcasted_iota(jnp.int32, (2, _NG, 8, 1), 1)

    vals, ixs = [], []
    for i in range(_R):
        w = dscr[...].reshape(2, _NG, 8, TQ)
        mb = m[:, None]
        eq = w == mb
        ixg = jnp.min(jnp.where(eq, g4, _NG - 1), axis=1)
        vals.append(m)
        ixs.append(ixg)
        if i < _R - 1:
            m = jnp.max(jnp.where(w < mb, w, -jnp.inf), axis=1)

    aio = lax.broadcasted_iota(jnp.int32, (2, 8, TQ), 0)
    sio = lax.broadcasted_iota(jnp.int32, (2, 8, TQ), 1)
    wv = jnp.concatenate([v.reshape(16, TQ) for v in vals], axis=0)
    nn = jnp.concatenate(
        [((aio * _NG + ix) * 8 + sio).reshape(16, TQ) for ix in ixs], axis=0)

    out = jnp.zeros((32, TQ), jnp.int32)
    orow = lax.broadcasted_iota(jnp.int32, (32, TQ), 0)
    base = b * N
    for j in range(K + 1):
        mm = jnp.max(wv, axis=0, keepdims=True)
        eq2 = wv == mm
        sel = jnp.min(jnp.where(eq2, nn, jnp.int32(1 << 30)),
                      axis=0, keepdims=True)
        if j > 0:
            out = jnp.where(orow == j - 1,
                            jnp.broadcast_to(sel + base, (32, TQ)), out)
        if j < K:
            wv = jnp.where(eq2 & (nn == sel), -jnp.inf, wv)
    idx_ref[...] = jnp.transpose(out, (1, 0))[:, :K]


B2 = B // 2


def _knn_call(xc, xq, pT, h):
    b0 = h * B2
    qn = N // TQ
    return pl.pallas_call(
        _knn_body,
        grid=(B2, qn),
        in_specs=[
            pl.BlockSpec((1, N, 8), lambda b, q: (b + b0, 0, 0)),
            pl.BlockSpec((1, 8, TQ), lambda b, q: (b + b0, 0, q)),
            pl.BlockSpec((TQ, C), lambda b, q: ((b + b0) * qn + q, 0)),
        ],
        out_specs=[
            pl.BlockSpec((TQ, K), lambda b, q: (b * qn + q, 0)),
            pl.BlockSpec((TQ, C), lambda b, q: (b * qn + q, 0)),
        ],
        out_shape=[
            jax.ShapeDtypeStruct((B2 * N, K), jnp.int32),
            jax.ShapeDtypeStruct((B2 * N, C), jnp.float32),
        ],
        scratch_shapes=[pltpu.VMEM((N, TQ), jnp.float32)],
    )(xq, xc, pT)


_NW = 32
_NIH = NUM_IDX // 2
_BPW = _NIH // _NW
_CH = 128
_NCHUNK = _BPW // _CH
_PPC = _CH // K
_NB = 4


def _sc_gather_sum(lpT, gidx):
    mesh = plsc.VectorSubcoreMesh(core_axis_name="c", subcore_axis_name="s")

    @functools.partial(
        pl.kernel,
        mesh=mesh,
        out_type=jax.ShapeDtypeStruct((B2 * N, C), jnp.float32),
        scratch_types=[
            pltpu.VMEM((_NB, _CH), jnp.int32),
            pltpu.VMEM((_NB, _CH, C), jnp.float32),
            pltpu.VMEM_SHARED((16, _NB, _PPC, C), jnp.float32),
            pltpu.VMEM((_PPC, C), jnp.float32),
            pltpu.VMEM((_CH,), jnp.int32),
        ] + [pltpu.SemaphoreType.DMA] * (2 * _NB),
    )
    def k(lp_hbm, idx_hbm, out_hbm, idx_v, rows_v, acc_sh, zeros_v, seg_v,
          *sems):
        gsem = sems[:_NB]
        osem = sems[_NB:]
        sid = lax.axis_index("s")
        wid = sid * 2 + lax.axis_index("c")
        ibase = wid * _BPW
        pbase = wid * (_BPW // K)
        for r in range(_PPC):
            seg_v[pl.ds(r * K, K)] = jnp.full((K,), r, jnp.int32)
            for g in range(C // 16):
                zeros_v[r, pl.ds(g * 16, 16)] = jnp.zeros((16,), jnp.float32)
        for s in range(_NB):
            pltpu.sync_copy(idx_hbm.at[pl.ds(ibase + s * _CH, _CH)],
                            idx_v.at[s])
            pltpu.async_copy(lp_hbm.at[idx_v.at[s]], rows_v.at[s], gsem[s])

        @pl.loop(0, _NCHUNK, step=_NB)
        def _(t0):
            for s in range(_NB):
                t = t0 + s
                acc = acc_sh.at[sid, s]
                @pl.when(t >= _NB)
                def _():
                    pltpu.make_async_copy(
                        acc, out_hbm.at[pl.ds(pbase, _PPC)],
                        osem[s]).wait()
                pltpu.sync_copy(zeros_v, acc)
                pltpu.make_async_copy(lp_hbm.at[pl.ds(0, _CH)], rows_v.at[s],
                                      gsem[s]).wait()
                pltpu.sync_copy(rows_v.at[s], acc.at[seg_v], add=True)
                pltpu.async_copy(acc,
                                 out_hbm.at[pl.ds(pbase + t * _PPC, _PPC)],
                                 osem[s])
                @pl.when(t + _NB < _NCHUNK)
                def _():
                    pltpu.sync_copy(
                        idx_hbm.at[pl.ds(ibase + (t + _NB) * _CH, _CH)],
                        idx_v.at[s])
                    pltpu.async_copy(lp_hbm.at[idx_v.at[s]], rows_v.at[s],
                                     gsem[s])

        for s in range(_NB):
            pltpu.make_async_copy(acc_sh.at[sid, s],
                                  out_hbm.at[pl.ds(pbase, _PPC)],
                                  osem[s]).wait()

    return k(lpT, gidx)


_TPB = N // TN


def _mm_body(p_ref, ns0_ref, ns1_ref, w0_ref, w1_ref, w2_ref, w3_ref,
             b0_ref, b1_ref, b2_ref, b3_ref, out_ref):
    i = pl.program_id(0)
    p = p_ref[...]
    lp = jnp.where(p >= 0, p, 0.01 * p)
    ns = jnp.where(i < (B2 * N // TN), ns0_ref[...], ns1_ref[...])
    t1 = (jnp.dot(lp, w0_ref[...], preferred_element_type=jnp.float32)
          + b0_ref[...]
          + jnp.dot(ns, w1_ref[...], preferred_element_type=jnp.float32)
          + K * b1_ref[...]) * (1.0 / (K + 1)) + p
    lt1 = jnp.where(t1 >= 0, t1, 0.01 * t1)
    w23 = w2_ref[...] + w3_ref[...]
    res = (jnp.dot(lt1, w23, preferred_element_type=jnp.float32)
           + (b2_ref[...] + b3_ref[...])) * 0.5 + t1
    out_ref[0] = jnp.transpose(res, (1, 0))


def _mm_call(pT, ns0, ns1, w0t, w1t, w2t, w3t, b0, b1, b2, b3):
    nh = B2 * N // TN
    wspec = pl.BlockSpec((C, C), lambda i: (0, 0))
    bspec = pl.BlockSpec((1, C), lambda i: (0, 0))
    hspec0 = pl.BlockSpec((TN, C), lambda i: (jnp.minimum(i, nh - 1), 0))
    hspec1 = pl.BlockSpec((TN, C),
                          lambda i: (jnp.maximum(i - nh, 0), 0))
    return pl.pallas_call(
        _mm_body,
        grid=(B * N // TN,),
        in_specs=[
            pl.BlockSpec((TN, C), lambda i: (i, 0)),
            hspec0, hspec1,
            wspec, wspec, wspec, wspec,
            bspec, bspec, bspec, bspec,
        ],
        out_specs=pl.BlockSpec((1, C, TN),
                               lambda i: (i // _TPB, 0, i % _TPB)),
        out_shape=jax.ShapeDtypeStruct((B, C, N), jnp.float32),
    )(pT, ns0, ns1, w0t, w1t, w2t, w3t, b0, b1, b2, b3)


def kernel(xyz, points, W0, b0, W1, b1, W2, b2, W3, b3):
    xt = jnp.transpose(xyz, (0, 2, 1))
    sq = jnp.sum(xt * xt, axis=2, keepdims=True)
    one = jnp.ones((B, N, 1), jnp.float32)
    zero = jnp.zeros((B, N, 3), jnp.float32)
    xq = jnp.concatenate([xt, sq, one, zero], axis=2)
    xc = jnp.transpose(
        jnp.concatenate([-2.0 * xt, one, sq, zero], axis=2), (0, 2, 1))
    pT = jnp.transpose(points, (0, 2, 1)).reshape(B * N, C)
    args = (W0.T, W1.T, W2.T, W3.T, b0.reshape(1, C), b1.reshape(1, C),
            b2.reshape(1, C), b3.reshape(1, C))
    nss = []
    for h in range(2):
        idx_h, lp_h = _knn_call(xc, xq, pT, h)
        nss.append(_sc_gather_sum(lp_h, idx_h.reshape(_NIH)))
    return _mm_call(pT, nss[0], nss[1], *args)
